# baseline (speedup 1.0000x reference)
import functools

import jax
import jax.numpy as jnp
from jax import lax
from jax.experimental import pallas as pl
from jax.experimental.pallas import tpu as pltpu
from jax.experimental.pallas import tpu_sc as plsc

EMBED = 64
CHUNK = 512


@functools.lru_cache(maxsize=None)
def _make_gather(n_total: int):
    info = plsc.get_sparse_core_info()
    nw = info.num_cores * info.num_subcores
    per_w = n_total // nw
    assert per_w * nw == n_total and per_w % CHUNK == 0
    n_chunks = per_w // CHUNK
    mesh = plsc.VectorSubcoreMesh(core_axis_name="c", subcore_axis_name="s")

    @functools.partial(
        pl.kernel,
        mesh=mesh,
        out_type=jax.ShapeDtypeStruct((n_total, EMBED), jnp.float32),
        scratch_types=[
            pltpu.VMEM((CHUNK,), jnp.int32),
            pltpu.VMEM((CHUNK, EMBED), jnp.float32),
            pltpu.SemaphoreType.DMA,
        ],
        compiler_params=pltpu.CompilerParams(use_tc_tiling_on_sc=False),
    )
    def gather_kernel(idx_hbm, table_hbm, out_hbm, idx_v, rows_v, sem):
        wid = lax.axis_index("s") * info.num_cores + lax.axis_index("c")
        base = wid * per_w

        def body(i, carry):
            off = base + i * CHUNK
            pltpu.sync_copy(idx_hbm.at[pl.ds(off, CHUNK)], idx_v)
            pltpu.async_copy(table_hbm.at[idx_v], rows_v, sem).wait()
            pltpu.sync_copy(rows_v, out_hbm.at[pl.ds(off, CHUNK)])
            return carry

        lax.fori_loop(0, n_chunks, body, 0)

    return gather_kernel


def kernel(x, table):
    b, l = x.shape
    flat = x.reshape(b * l).astype(jnp.int32)
    out = _make_gather(b * l)(flat, table)
    return out.reshape(b, l, EMBED)


# speedup vs baseline: 1.0346x; 1.0346x over previous
import functools

import jax
import jax.numpy as jnp
from jax import lax
from jax.experimental import pallas as pl
from jax.experimental.pallas import tpu as pltpu
from jax.experimental.pallas import tpu_sc as plsc

EMBED = 64
CHUNK = 512
NBUF = 2


@functools.lru_cache(maxsize=None)
def _make_gather(n_total: int):
    info = plsc.get_sparse_core_info()
    nw = info.num_cores * info.num_subcores
    per_w = n_total // nw
    assert per_w * nw == n_total and per_w % (CHUNK * NBUF) == 0
    n_rounds = per_w // (CHUNK * NBUF)
    assert n_rounds >= 2
    mesh = plsc.VectorSubcoreMesh(core_axis_name="c", subcore_axis_name="s")

    @functools.partial(
        pl.kernel,
        mesh=mesh,
        out_type=jax.ShapeDtypeStruct((n_total, EMBED), jnp.float32),
        scratch_types=[
            pltpu.VMEM((per_w,), jnp.int32),
            pltpu.VMEM((NBUF, CHUNK, EMBED), jnp.float32),
            pltpu.SemaphoreType.DMA((NBUF,)),
            pltpu.SemaphoreType.DMA((NBUF,)),
        ],
        compiler_params=pltpu.CompilerParams(use_tc_tiling_on_sc=False),
    )
    def gather_kernel(idx_hbm, table_hbm, out_hbm, idx_all, rows, gsem, ssem):
        wid = lax.axis_index("s") * info.num_cores + lax.axis_index("c")
        base = wid * per_w

        def idx_slice(c):
            return idx_all.at[pl.ds(c * CHUNK, CHUNK)]

        def out_slice(c):
            return out_hbm.at[pl.ds(base + c * CHUNK, CHUNK)]

        pltpu.sync_copy(idx_hbm.at[pl.ds(base, per_w)], idx_all)
        for b in range(NBUF):
            pltpu.async_copy(table_hbm.at[idx_slice(b)], rows.at[b], gsem.at[b])

        def body(g, carry):
            c0 = g * NBUF
            for b in range(NBUF):
                pltpu.make_async_copy(
                    table_hbm.at[idx_slice(c0 + b)], rows.at[b], gsem.at[b]
                ).wait()
                pltpu.async_copy(rows.at[b], out_slice(c0 + b), ssem.at[b])
            for b in range(NBUF):
                pltpu.make_async_copy(
                    rows.at[b], out_slice(c0 + b), ssem.at[b]
                ).wait()
                pltpu.async_copy(
                    table_hbm.at[idx_slice(c0 + NBUF + b)], rows.at[b], gsem.at[b]
                )
            return carry

        lax.fori_loop(0, n_rounds - 1, body, 0)

        cf = (n_rounds - 1) * NBUF
        for b in range(NBUF):
            pltpu.make_async_copy(
                table_hbm.at[idx_slice(cf + b)], rows.at[b], gsem.at[b]
            ).wait()
            pltpu.async_copy(rows.at[b], out_slice(cf + b), ssem.at[b])
        for b in range(NBUF):
            pltpu.make_async_copy(rows.at[b], out_slice(cf + b), ssem.at[b]).wait()

    return gather_kernel


def kernel(x, table):
    b, l = x.shape
    flat = x.reshape(b * l).astype(jnp.int32)
    out = _make_gather(b * l)(flat, table)
    return out.reshape(b, l, EMBED)


# speedup vs baseline: 1.0640x; 1.0284x over previous
import functools

import jax
import jax.numpy as jnp
from jax import lax
from jax.experimental import pallas as pl
from jax.experimental.pallas import tpu as pltpu
from jax.experimental.pallas import tpu_sc as plsc

EMBED = 64
CHUNK = 512
NBUF = 2


@functools.lru_cache(maxsize=None)
def _make_gather(n_total: int):
    info = plsc.get_sparse_core_info()
    nw = info.num_cores * info.num_subcores
    per_w = n_total // nw
    assert per_w * nw == n_total and per_w % (CHUNK * NBUF) == 0
    n_rounds = per_w // (CHUNK * NBUF)
    assert n_rounds >= 2
    mesh = plsc.VectorSubcoreMesh(core_axis_name="c", subcore_axis_name="s")

    @functools.partial(
        pl.kernel,
        mesh=mesh,
        out_type=jax.ShapeDtypeStruct((n_total, EMBED), jnp.float32),
        scratch_types=[
            pltpu.VMEM((per_w,), jnp.int32),
            pltpu.VMEM((NBUF, CHUNK, EMBED), jnp.float32),
            pltpu.SemaphoreType.DMA((NBUF,)),
            pltpu.SemaphoreType.DMA((NBUF,)),
        ],
        compiler_params=pltpu.CompilerParams(use_tc_tiling_on_sc=False),
    )
    def gather_kernel(idx_hbm, table_hbm, out_hbm, idx_all, rows, gsem, ssem):
        wid = lax.axis_index("s") * info.num_cores + lax.axis_index("c")
        base = wid * per_w

        def idx_slice(c):
            return idx_all.at[pl.ds(c * CHUNK, CHUNK)]

        def out_slice(c):
            return out_hbm.at[pl.ds(base + c * CHUNK, CHUNK)]

        pltpu.sync_copy(idx_hbm.at[pl.ds(base, per_w)], idx_all)
        for b in range(NBUF):
            pltpu.async_copy(table_hbm.at[idx_slice(b)], rows.at[b], gsem.at[b])

        def body(g, carry):
            c0 = g * NBUF
            for b in range(NBUF):
                pltpu.make_async_copy(
                    table_hbm.at[idx_slice(c0 + b)], rows.at[b], gsem.at[b]
                ).wait()
                pltpu.async_copy(rows.at[b], out_slice(c0 + b), ssem.at[b])
            for b in range(NBUF):
                pltpu.make_async_copy(
                    rows.at[b], out_slice(c0 + b), ssem.at[b]
                ).wait()
                pltpu.async_copy(
                    table_hbm.at[idx_slice(c0 + NBUF + b)], rows.at[b], gsem.at[b]
                )
            return carry

        lax.fori_loop(0, n_rounds - 1, body, 0)

        cf = (n_rounds - 1) * NBUF
        for b in range(NBUF):
            pltpu.make_async_copy(
                table_hbm.at[idx_slice(cf + b)], rows.at[b], gsem.at[b]
            ).wait()
            pltpu.async_copy(rows.at[b], out_slice(cf + b), ssem.at[b])
        for b in range(NBUF):
            pltpu.make_async_copy(rows.at[b], out_slice(cf + b), ssem.at[b]).wait()

    return gather_kernel


def kernel(x, table):
    b, l = x.shape
    flat = x.T.reshape(b * l).astype(jnp.int32)
    out = _make_gather(b * l)(flat, table)
    return out.reshape(l, b, EMBED).transpose(1, 0, 2)
